# Initial kernel scaffold; baseline (speedup 1.0000x reference)
import jax
import jax.numpy as jnp
from jax.experimental import pallas as pl


def kernel(box_labels, ent_emb, rel_emb, norm_vec):
    raise NotImplementedError("write your pallas kernel here")


# speedup vs baseline: 6.3740x; 6.3740x over previous
import jax
import jax.numpy as jnp
from jax import lax
from jax.experimental import pallas as pl
from jax.experimental.pallas import tpu as pltpu

_N_H = 8
_N = 64
_NUM_CLS = 117
_NUM_OBJ = 80
_HUMAN = 49
_DIM = 300
_PAIRS = _N_H * _N - _N_H
_G = 8
_STEPS = _PAIRS // _G


def _l2n(x):
    return x / jnp.maximum(jnp.sqrt(jnp.sum(x * x, axis=-1, keepdims=True)),
                           1e-12)


def _body(lab_ref, ent_ref, rel_ref, nv_ref,
          h_out, r_out, w_out, t_out, s_out,
          tn_s, hp_s, rn_s, wn_s):
    i = pl.program_id(0)

    @pl.when(i == 0)
    def _prep():
        lab = jnp.where(lax.broadcasted_iota(jnp.int32, (_N, 1), 0) < _N_H,
                        _HUMAN, lab_ref[...])
        oh = (lab == lax.broadcasted_iota(jnp.int32, (_N, _NUM_OBJ), 1)
              ).astype(jnp.float32)
        ent = ent_ref[...]
        tn_s[...] = _l2n(jnp.dot(oh, ent, preferred_element_type=jnp.float32))
        hn = _l2n(ent[_HUMAN:_HUMAN + 1, :])
        wn = _l2n(nv_ref[...])
        wn_s[...] = wn
        rn_s[...] = _l2n(rel_ref[...])
        hp_s[...] = hn - jnp.sum(hn * wn, axis=-1, keepdims=True) * wn

    k = i * _G + lax.broadcasted_iota(jnp.int32, (_G, 1), 0)
    x = k // (_N - 1)
    j = k % (_N - 1)
    y = j + (j >= x).astype(jnp.int32)
    oh8 = (y == lax.broadcasted_iota(jnp.int32, (_G, _N), 1)
           ).astype(jnp.float32)
    tn8 = jnp.dot(oh8, tn_s[...], preferred_element_type=jnp.float32)
    wn = wn_s[...]
    rn = rn_s[...]
    hp = hp_s[...]
    dot8 = lax.dot_general(tn8, wn, (((1,), (1,)), ((), ())),
                           preferred_element_type=jnp.float32)
    tp8 = tn8[:, None, :] - dot8[:, :, None] * wn[None, :, :]
    a = hp + rn
    diff = a[None, :, :] - tp8
    s_out[...] = jnp.sqrt(jnp.sum(diff * diff, axis=-1))
    h_out[...] = jnp.broadcast_to(hp[None], (_G, _NUM_CLS, _DIM))
    r_out[...] = jnp.broadcast_to(rn[None], (_G, _NUM_CLS, _DIM))
    w_out[...] = jnp.broadcast_to(wn[None], (_G, _NUM_CLS, _DIM))
    t_out[...] = tp8


def kernel(box_labels, ent_emb, rel_emb, norm_vec):
    big = jax.ShapeDtypeStruct((_PAIRS, _NUM_CLS, _DIM), jnp.float32)
    out_shapes = (big, big, big, big,
                  jax.ShapeDtypeStruct((_PAIRS, _NUM_CLS), jnp.float32))
    big_spec = pl.BlockSpec((_G, _NUM_CLS, _DIM), lambda i: (i, 0, 0))
    return pl.pallas_call(
        _body,
        grid=(_STEPS,),
        in_specs=[
            pl.BlockSpec((_N, 1), lambda i: (0, 0)),
            pl.BlockSpec((_NUM_OBJ, _DIM), lambda i: (0, 0)),
            pl.BlockSpec((_NUM_CLS, _DIM), lambda i: (0, 0)),
            pl.BlockSpec((_NUM_CLS, _DIM), lambda i: (0, 0)),
        ],
        out_specs=(big_spec, big_spec, big_spec, big_spec,
                   pl.BlockSpec((_G, _NUM_CLS), lambda i: (i, 0))),
        out_shape=out_shapes,
        scratch_shapes=[
            pltpu.VMEM((_N, _DIM), jnp.float32),
            pltpu.VMEM((_NUM_CLS, _DIM), jnp.float32),
            pltpu.VMEM((_NUM_CLS, _DIM), jnp.float32),
            pltpu.VMEM((_NUM_CLS, _DIM), jnp.float32),
        ],
    )(box_labels.reshape(_N, 1), ent_emb, rel_emb, norm_vec)
